# baseline (speedup 1.0000x reference)
import jax
import jax.numpy as jnp
from jax.experimental import pallas as pl
from jax.experimental.pallas import tpu as pltpu

_SR = 4
_D = 2 * _SR + 1
_NC = _D * _D


def _cv_body(x1_ref, x2a_ref, x2b_ref, o_ref, ws_a, ws_b, x1s):
    C, Hb, W = x1_ref.shape[1], x1_ref.shape[2], x1_ref.shape[3]
    inv = jnp.float32(1.0 / _NC)
    bufs = (ws_a, ws_b)
    for hg in range(Hb // 8):
        for cg in range(C // 8):
            tile = x1_ref[0, cg * 8:(cg + 1) * 8, hg * 8:(hg + 1) * 8, :]
            x1s[hg * 8:(hg + 1) * 8, cg * 8:(cg + 1) * 8, :] = (
                jnp.swapaxes(tile, 0, 1) * inv)
    ws_a[:Hb] = x2a_ref[0, :, :, 0:W]
    ws_a[Hb:] = x2b_ref[0, : 2 * _SR, :, 0:W]
    for dj in range(_D):
        cur = bufs[dj % 2]
        nxt = bufs[(dj + 1) % 2]
        stage_next = dj + 1 < _D

        def hbody(h, carry, cur=cur, nxt=nxt, stage_next=stage_next, dj=dj):
            x1p = x1s[h]
            for di in range(_D):
                k = (_D * (_SR - di) + (_SR - dj)) % _NC
                o_ref[0, k, h] = jnp.sum(x1p * cur[h + di], axis=0)
            if stage_next:
                nxt[h] = x2a_ref[0, h, :, dj + 1:dj + 1 + W]
            return carry

        jax.lax.fori_loop(0, Hb, hbody, 0, unroll=8)
        if stage_next:
            nxt[Hb:] = x2b_ref[0, : 2 * _SR, :, dj + 1:dj + 1 + W]


def kernel(x1, x2):
    B, C, H, W = x1.shape
    Hb = 64 if H % 64 == 0 else H
    nH = H // Hb
    Hp = (nH + 1) * Hb
    x2t = jnp.pad(jnp.transpose(x2, (0, 2, 1, 3)),
                  ((0, 0), (_SR, Hp - H - _SR), (0, 0), (_SR, _SR)))
    Wp = W + 2 * _SR

    out = pl.pallas_call(
        _cv_body,
        out_shape=jax.ShapeDtypeStruct((B, _NC, H, W), x1.dtype),
        grid=(B, nH),
        in_specs=[
            pl.BlockSpec((1, C, Hb, W), lambda b, h: (b, 0, h, 0)),
            pl.BlockSpec((1, Hb, C, Wp), lambda b, h: (b, h, 0, 0)),
            pl.BlockSpec((1, Hb, C, Wp), lambda b, h: (b, h + 1, 0, 0)),
        ],
        out_specs=pl.BlockSpec((1, _NC, Hb, W), lambda b, h: (b, 0, h, 0)),
        scratch_shapes=[
            pltpu.VMEM((Hb + 2 * _SR, C, W), jnp.float32),
            pltpu.VMEM((Hb + 2 * _SR, C, W), jnp.float32),
            pltpu.VMEM((Hb, C, W), jnp.float32),
        ],
        compiler_params=pltpu.CompilerParams(
            dimension_semantics=("parallel", "arbitrary"),
            vmem_limit_bytes=56 * 1024 * 1024,
        ),
        name="cost_volume_t",
    )(x1, x2t, x2t)
    return out


# speedup vs baseline: 1.0252x; 1.0252x over previous
import jax
import jax.numpy as jnp
from jax.experimental import pallas as pl
from jax.experimental.pallas import tpu as pltpu

_SR = 4
_D = 2 * _SR + 1
_NC = _D * _D


def _cv_body(x1_ref, x2a_ref, x2b_ref, o_ref, ws_a, ws_b, x1s):
    C, Hb, W = x1_ref.shape[1], x1_ref.shape[2], x1_ref.shape[3]
    inv = jnp.float32(1.0 / _NC)
    bufs = (ws_a, ws_b)
    for hg in range(Hb // 8):
        for cg in range(C // 8):
            tile = x1_ref[0, cg * 8:(cg + 1) * 8, hg * 8:(hg + 1) * 8, :]
            x1s[hg * 8:(hg + 1) * 8, cg * 8:(cg + 1) * 8, :] = (
                jnp.swapaxes(tile, 0, 1) * inv)
    ws_a[:Hb] = x2a_ref[0, :, :, 0:W]
    ws_a[Hb:] = x2b_ref[0, : 2 * _SR, :, 0:W]
    for dj in range(_D):
        cur = bufs[dj % 2]
        nxt = bufs[(dj + 1) % 2]
        stage_next = dj + 1 < _D

        def hbody(h, carry, cur=cur, nxt=nxt, stage_next=stage_next, dj=dj):
            x1p = x1s[h]
            for di in range(_D):
                k = (_D * (_SR - di) + (_SR - dj)) % _NC
                o_ref[0, k, h] = jnp.sum(x1p * cur[h + di], axis=0)
            if stage_next:
                nxt[h] = x2a_ref[0, h, :, dj + 1:dj + 1 + W]
            return carry

        jax.lax.fori_loop(0, Hb, hbody, 0, unroll=4)
        if stage_next:
            nxt[Hb:] = x2b_ref[0, : 2 * _SR, :, dj + 1:dj + 1 + W]


def kernel(x1, x2):
    B, C, H, W = x1.shape
    Hb = 64 if H % 64 == 0 else H
    nH = H // Hb
    Hp = (nH + 1) * Hb
    x2t = jnp.pad(jnp.transpose(x2, (0, 2, 1, 3)),
                  ((0, 0), (_SR, Hp - H - _SR), (0, 0), (_SR, _SR)))
    Wp = W + 2 * _SR

    out = pl.pallas_call(
        _cv_body,
        out_shape=jax.ShapeDtypeStruct((B, _NC, H, W), x1.dtype),
        grid=(B, nH),
        in_specs=[
            pl.BlockSpec((1, C, Hb, W), lambda b, h: (b, 0, h, 0)),
            pl.BlockSpec((1, Hb, C, Wp), lambda b, h: (b, h, 0, 0)),
            pl.BlockSpec((1, Hb, C, Wp), lambda b, h: (b, h + 1, 0, 0)),
        ],
        out_specs=pl.BlockSpec((1, _NC, Hb, W), lambda b, h: (b, 0, h, 0)),
        scratch_shapes=[
            pltpu.VMEM((Hb + 2 * _SR, C, W), jnp.float32),
            pltpu.VMEM((Hb + 2 * _SR, C, W), jnp.float32),
            pltpu.VMEM((Hb, C, W), jnp.float32),
        ],
        compiler_params=pltpu.CompilerParams(
            dimension_semantics=("parallel", "arbitrary"),
            vmem_limit_bytes=56 * 1024 * 1024,
        ),
        name="cost_volume_t",
    )(x1, x2t, x2t)
    return out


# speedup vs baseline: 1.0822x; 1.0556x over previous
import jax
import jax.numpy as jnp
from jax.experimental import pallas as pl
from jax.experimental.pallas import tpu as pltpu

_SR = 4
_D = 2 * _SR + 1
_NC = _D * _D


def _cv_body(x1_ref, x2a_ref, x2b_ref, o_ref, ws_a, ws_b, x1s):
    C, Hb, W = x1_ref.shape[1], x1_ref.shape[2], x1_ref.shape[3]
    inv = jnp.float32(1.0 / _NC)
    bufs = (ws_a, ws_b)
    for hg in range(Hb // 8):
        for cg in range(C // 8):
            tile = x1_ref[0, cg * 8:(cg + 1) * 8, hg * 8:(hg + 1) * 8, :]
            x1s[hg * 8:(hg + 1) * 8, cg * 8:(cg + 1) * 8, :] = (
                jnp.swapaxes(tile, 0, 1) * inv)
    ws_a[:Hb] = x2a_ref[0, :, :, 0:W]
    ws_a[Hb:] = x2b_ref[0, : 2 * _SR, :, 0:W]
    c_slices = ((0, 3 * C // 8), (3 * C // 8, 6 * C // 8), (6 * C // 8, C))
    for dj in range(_D):
        cur = bufs[dj % 2]
        nxt = bufs[(dj + 1) % 2]
        stage_next = dj + 1 < _D
        for g, di0 in enumerate(range(0, _D, 3)):
            dis = (di0, di0 + 1, di0 + 2)
            c0, c1 = c_slices[g]

            def hbody(h, carry, dis=dis, cur=cur, nxt=nxt,
                      stage_next=stage_next, dj=dj, c0=c0, c1=c1):
                x1p = x1s[h]
                for di in dis:
                    k = (_D * (_SR - di) + (_SR - dj)) % _NC
                    o_ref[0, k, h] = jnp.sum(x1p * cur[h + di], axis=0)
                if stage_next:
                    nxt[h, c0:c1] = x2a_ref[0, h, c0:c1, dj + 1:dj + 1 + W]
                return carry

            jax.lax.fori_loop(0, Hb, hbody, 0, unroll=8)
            if stage_next and g == 0:
                nxt[Hb:] = x2b_ref[0, : 2 * _SR, :, dj + 1:dj + 1 + W]


def kernel(x1, x2):
    B, C, H, W = x1.shape
    Hb = 64 if H % 64 == 0 else H
    nH = H // Hb
    Hp = (nH + 1) * Hb
    x2t = jnp.pad(jnp.transpose(x2, (0, 2, 1, 3)),
                  ((0, 0), (_SR, Hp - H - _SR), (0, 0), (_SR, _SR)))
    Wp = W + 2 * _SR

    out = pl.pallas_call(
        _cv_body,
        out_shape=jax.ShapeDtypeStruct((B, _NC, H, W), x1.dtype),
        grid=(B, nH),
        in_specs=[
            pl.BlockSpec((1, C, Hb, W), lambda b, h: (b, 0, h, 0)),
            pl.BlockSpec((1, Hb, C, Wp), lambda b, h: (b, h, 0, 0)),
            pl.BlockSpec((1, Hb, C, Wp), lambda b, h: (b, h + 1, 0, 0)),
        ],
        out_specs=pl.BlockSpec((1, _NC, Hb, W), lambda b, h: (b, 0, h, 0)),
        scratch_shapes=[
            pltpu.VMEM((Hb + 2 * _SR, C, W), jnp.float32),
            pltpu.VMEM((Hb + 2 * _SR, C, W), jnp.float32),
            pltpu.VMEM((Hb, C, W), jnp.float32),
        ],
        compiler_params=pltpu.CompilerParams(
            dimension_semantics=("parallel", "arbitrary"),
            vmem_limit_bytes=56 * 1024 * 1024,
        ),
        name="cost_volume_t",
    )(x1, x2t, x2t)
    return out
